# baseline (speedup 1.0000x reference)
import functools

import jax
import jax.numpy as jnp
from jax import lax
from jax.experimental import pallas as pl
from jax.experimental.pallas import tpu as pltpu
from jax.experimental.pallas import tpu_sc as plsc

_CAP_FRAC = 0.5
_AUX_W = 0.01

_NC = 2
_NS = 16
_NW = _NC * _NS


def _matvec_body(x_ref, w_ref, out_ref):
    out_ref[...] = jax.lax.dot_general(
        x_ref[...].astype(jnp.bfloat16), w_ref[...].astype(jnp.bfloat16),
        dimension_numbers=(((1,), (0,)), ((), ())),
        preferred_element_type=jnp.float32,
    )


def _tc_logits(x2, w2, blk=1024):
    n, d = x2.shape
    return pl.pallas_call(
        _matvec_body,
        grid=(n // blk,),
        in_specs=[
            pl.BlockSpec((blk, d), lambda i: (i, 0)),
            pl.BlockSpec((d, 1), lambda i: (0, 0)),
        ],
        out_specs=pl.BlockSpec((blk, 1), lambda i: (i, 0)),
        out_shape=jax.ShapeDtypeStruct((n, 1), jnp.float32),
    )(x2, w2)


def _rne_bf16(v):
    t = v * jnp.float32(65537.0)
    return t - (t - v)


def _sc_logits(x_sc, w, ch=32):
    n, d = x_sc.shape
    rows = n // _NW
    nch = rows // ch
    nv = d // 16
    mesh = plsc.VectorSubcoreMesh(core_axis_name="c", subcore_axis_name="s")

    @functools.partial(
        pl.kernel, mesh=mesh,
        out_type=jax.ShapeDtypeStruct((n,), jnp.float32),
        scratch_types=[
            pltpu.VMEM((d,), jnp.float32),
            pltpu.VMEM((ch, d), jnp.float32),
            pltpu.VMEM((rows,), jnp.float32),
            pltpu.SemaphoreType.DMA,
        ],
    )
    def k(x_hbm, w_hbm, out_hbm, w_v, xbuf, outb, sem):
        wid = lax.axis_index("s") * _NC + lax.axis_index("c")
        base = wid * rows
        pltpu.sync_copy(w_hbm, w_v)

        def round_w(i, carry):
            w_v[pl.ds(i * 16, 16)] = _rne_bf16(w_v[pl.ds(i * 16, 16)])
            return carry

        lax.fori_loop(0, nv, round_w, 0)
        lane = lax.iota(jnp.int32, 16)

        for c in range(nch):
            pltpu.async_copy(
                x_hbm.at[pl.ds(base + c * ch, ch), :], xbuf, sem).wait()

            def do_group(g, carry):
                ov = jnp.zeros((16,), jnp.float32)
                for r16 in range(16):
                    r = g * 16 + r16

                    def dstep(i, acc):
                        xv = _rne_bf16(xbuf[r, pl.ds(i * 16, 16)])
                        return acc + xv * w_v[pl.ds(i * 16, 16)]

                    s = lax.fori_loop(
                        0, nv, dstep, jnp.zeros((16,), jnp.float32))
                    for kk in (8, 4, 2, 1):
                        idx = (lane + jnp.int32(kk)) % jnp.int32(16)
                        s = s + s.at[idx].get(mode="promise_in_bounds")
                    ov = jnp.where(lane == jnp.int32(r16), s, ov)
                outb[pl.ds(c * ch + g * 16, 16)] = ov
                return carry

            lax.fori_loop(0, ch // 16, do_group, 0)

        pltpu.sync_copy(outb, out_hbm.at[pl.ds(base, rows)])

    return k(x_sc, w)


def _make_mask_body(cap):
    def _mask_body(logits_ref, noise_ref, mask_ref, aux_ref):
        lg = logits_ref[...]
        b_, l_ = lg.shape
        noisy = lg + noise_ref[...]
        ui = jax.lax.bitcast_convert_type(noisy, jnp.int32)
        ikey = jnp.where(ui < 0, ui ^ jnp.int32(0x7FFFFFFF), ui)
        msb = jnp.int32(-2147483648)

        def step(i, u):
            cand = u | jax.lax.shift_left(jnp.int32(1), jnp.int32(31) - i)
            cand_s = cand ^ msb
            cnt = jnp.sum((ikey >= cand_s).astype(jnp.int32), axis=1,
                          keepdims=True)
            return jnp.where(cnt >= cap, cand, u)

        u = jax.lax.fori_loop(0, 32, step, jnp.zeros((b_, 1), jnp.int32))
        t = u ^ msb
        gt = ikey > t
        eq = ikey == t
        n_gt = jnp.sum(gt.astype(jnp.int32), axis=1, keepdims=True)
        rem = cap - n_gt
        c = eq.astype(jnp.int32)
        sh = 1
        while sh < l_:
            c = c + jnp.concatenate(
                [jnp.zeros((b_, sh), jnp.int32), c[:, : l_ - sh]], axis=1)
            sh *= 2
        mask = gt | (eq & (c <= rem))
        mask_ref[...] = mask.astype(jnp.int32)

        probs = 1.0 / (1.0 + jnp.exp(-lg))
        rowmean = jnp.sum(probs, axis=1, keepdims=True) * (1.0 / l_)
        dev = rowmean - _CAP_FRAC
        aux_ref[...] = jnp.sum(dev * dev, axis=0, keepdims=True) * (_AUX_W / b_)

    return _mask_body


def _mask_and_aux(logits, noise, cap):
    b, l = logits.shape
    return pl.pallas_call(
        _make_mask_body(cap),
        in_specs=[
            pl.BlockSpec((b, l), lambda: (0, 0)),
            pl.BlockSpec((b, l), lambda: (0, 0)),
        ],
        out_specs=[
            pl.BlockSpec((b, l), lambda: (0, 0)),
            pl.BlockSpec((1, 1), lambda: (0, 0)),
        ],
        out_shape=[
            jax.ShapeDtypeStruct((b, l), jnp.int32),
            jax.ShapeDtypeStruct((1, 1), jnp.float32),
        ],
    )(logits, noise)


_SC_ROWS = 2048


def kernel(x, W):
    b, l, d = x.shape
    cap = max(1, int(l * _CAP_FRAC))
    n = b * l
    x2 = x.reshape(n, d)
    n_tc = n - _SC_ROWS
    logits_tc = _tc_logits(x2[:n_tc], W.reshape(d, 1))[:, 0]
    logits_sc = _sc_logits(x2[n_tc:], W.reshape(d))
    logits = jnp.concatenate([logits_tc, logits_sc]).reshape(b, l)
    noise = jax.random.normal(jax.random.key(1), (b, l), jnp.float32) * 0.1
    mask_i, aux = _mask_and_aux(logits, noise, cap)
    return mask_i.astype(jnp.bool_), logits, aux[0, 0]


# speedup vs baseline: 1.0022x; 1.0022x over previous
import functools

import jax
import jax.numpy as jnp
from jax import lax
from jax.experimental import pallas as pl
from jax.experimental.pallas import tpu as pltpu
from jax.experimental.pallas import tpu_sc as plsc

_CAP_FRAC = 0.5
_AUX_W = 0.01

_NC = 2
_NS = 16
_NW = _NC * _NS


def _matvec_body(x_ref, w_ref, out_ref):
    out_ref[...] = jax.lax.dot_general(
        x_ref[...].astype(jnp.bfloat16), w_ref[...].astype(jnp.bfloat16),
        dimension_numbers=(((1,), (0,)), ((), ())),
        preferred_element_type=jnp.float32,
    )


def _tc_logits(x2, w2, blk=1024):
    n, d = x2.shape
    return pl.pallas_call(
        _matvec_body,
        grid=(n // blk,),
        in_specs=[
            pl.BlockSpec((blk, d), lambda i: (i, 0)),
            pl.BlockSpec((d, 1), lambda i: (0, 0)),
        ],
        out_specs=pl.BlockSpec((blk, 1), lambda i: (i, 0)),
        out_shape=jax.ShapeDtypeStruct((n, 1), jnp.float32),
    )(x2, w2)


def _rne_bf16(v):
    t = v * jnp.float32(65537.0)
    return t - (t - v)


def _sc_logits(x_sc, w, ch=16, unroll=8):
    n, d = x_sc.shape
    rows = n // _NW
    nch = rows // ch
    nv = d // 16
    mesh = plsc.VectorSubcoreMesh(
        core_axis_name="c", subcore_axis_name="s", num_cores=_NC)

    @functools.partial(
        pl.kernel, mesh=mesh,
        out_type=jax.ShapeDtypeStruct((n,), jnp.float32),
        scratch_types=[
            pltpu.VMEM((d,), jnp.float32),
            pltpu.VMEM((2, ch, d), jnp.float32),
            pltpu.VMEM((rows,), jnp.float32),
            pltpu.SemaphoreType.DMA,
            pltpu.SemaphoreType.DMA,
        ],
    )
    def k(x_hbm, w_hbm, out_hbm, w_v, xbuf, outb, sem0, sem1):
        wid = lax.axis_index("s") * _NC + lax.axis_index("c")
        base = wid * rows
        pltpu.sync_copy(w_hbm, w_v)

        def round_w(i, carry):
            w_v[pl.ds(i * 16, 16)] = _rne_bf16(w_v[pl.ds(i * 16, 16)])
            return carry

        lax.fori_loop(0, nv, round_w, 0)
        lane = lax.iota(jnp.int32, 16)
        sems = (sem0, sem1)

        def start(c):
            return pltpu.async_copy(
                x_hbm.at[pl.ds(base + c * ch, ch), :],
                xbuf.at[c % 2], sems[c % 2])

        pending = start(0)
        for c in range(nch):
            nxt = start(c + 1) if c + 1 < nch else None
            pending.wait()
            pending = nxt

            def do_group(g, carry):
                ov = jnp.zeros((16,), jnp.float32)
                for r16 in range(16):
                    r = g * 16 + r16

                    def dstep(i, acc):
                        b0 = i * (16 * unroll)
                        for u in range(unroll):
                            xv = _rne_bf16(
                                xbuf[c % 2, r, pl.ds(b0 + u * 16, 16)])
                            acc = acc + xv * w_v[pl.ds(b0 + u * 16, 16)]
                        return acc

                    s = lax.fori_loop(
                        0, nv // unroll, dstep,
                        jnp.zeros((16,), jnp.float32))
                    for kk in (8, 4, 2, 1):
                        idx = (lane + jnp.int32(kk)) % jnp.int32(16)
                        s = s + s.at[idx].get(mode="promise_in_bounds")
                    ov = jnp.where(lane == jnp.int32(r16), s, ov)
                outb[pl.ds(c * ch + g * 16, 16)] = ov
                return carry

            lax.fori_loop(0, ch // 16, do_group, 0)

        pltpu.sync_copy(outb, out_hbm.at[pl.ds(base, rows)])

    return k(x_sc, w)


def _make_mask_body(cap):
    def _mask_body(logits_ref, noise_ref, mask_ref, aux_ref):
        lg = logits_ref[...]
        b_, l_ = lg.shape
        noisy = lg + noise_ref[...]
        ui = jax.lax.bitcast_convert_type(noisy, jnp.int32)
        ikey = jnp.where(ui < 0, ui ^ jnp.int32(0x7FFFFFFF), ui)
        msb = jnp.int32(-2147483648)

        def step(i, u):
            cand = u | jax.lax.shift_left(jnp.int32(1), jnp.int32(31) - i)
            cand_s = cand ^ msb
            cnt = jnp.sum((ikey >= cand_s).astype(jnp.int32), axis=1,
                          keepdims=True)
            return jnp.where(cnt >= cap, cand, u)

        u = jax.lax.fori_loop(0, 32, step, jnp.zeros((b_, 1), jnp.int32))
        t = u ^ msb
        gt = ikey > t
        eq = ikey == t
        n_gt = jnp.sum(gt.astype(jnp.int32), axis=1, keepdims=True)
        rem = cap - n_gt
        c = eq.astype(jnp.int32)
        sh = 1
        while sh < l_:
            c = c + jnp.concatenate(
                [jnp.zeros((b_, sh), jnp.int32), c[:, : l_ - sh]], axis=1)
            sh *= 2
        mask = gt | (eq & (c <= rem))
        mask_ref[...] = mask.astype(jnp.int32)

        probs = 1.0 / (1.0 + jnp.exp(-lg))
        rowmean = jnp.sum(probs, axis=1, keepdims=True) * (1.0 / l_)
        dev = rowmean - _CAP_FRAC
        aux_ref[...] = jnp.sum(dev * dev, axis=0, keepdims=True) * (_AUX_W / b_)

    return _mask_body


def _mask_and_aux(logits, noise, cap):
    b, l = logits.shape
    return pl.pallas_call(
        _make_mask_body(cap),
        in_specs=[
            pl.BlockSpec((b, l), lambda: (0, 0)),
            pl.BlockSpec((b, l), lambda: (0, 0)),
        ],
        out_specs=[
            pl.BlockSpec((b, l), lambda: (0, 0)),
            pl.BlockSpec((1, 1), lambda: (0, 0)),
        ],
        out_shape=[
            jax.ShapeDtypeStruct((b, l), jnp.int32),
            jax.ShapeDtypeStruct((1, 1), jnp.float32),
        ],
    )(logits, noise)


_SC_ROWS = 2048


def kernel(x, W):
    b, l, d = x.shape
    cap = max(1, int(l * _CAP_FRAC))
    n = b * l
    x2 = x.reshape(n, d)
    n_tc = n - _SC_ROWS
    logits_tc = _tc_logits(x2[:n_tc], W.reshape(d, 1))[:, 0]
    logits_sc = _sc_logits(x2[n_tc:], W.reshape(d))
    logits = jnp.concatenate([logits_tc, logits_sc]).reshape(b, l)
    noise = jax.random.normal(jax.random.key(1), (b, l), jnp.float32) * 0.1
    mask_i, aux = _mask_and_aux(logits, noise, cap)
    return mask_i.astype(jnp.bool_), logits, aux[0, 0]


# speedup vs baseline: 2.0601x; 2.0555x over previous
import functools

import jax
import jax.numpy as jnp
from jax import lax
from jax.experimental import pallas as pl
from jax.experimental.pallas import tpu as pltpu
from jax.experimental.pallas import tpu_sc as plsc

_CAP_FRAC = 0.5
_AUX_W = 0.01

_NC = 2
_NS = 16
_NW = _NC * _NS


def _matvec_body(x_ref, w_ref, out_ref):
    out_ref[...] = jax.lax.dot_general(
        x_ref[...].astype(jnp.bfloat16), w_ref[...].astype(jnp.bfloat16),
        dimension_numbers=(((1,), (0,)), ((), ())),
        preferred_element_type=jnp.float32,
    )


def _tc_logits(x2, w2, n_tc, blk=1024):
    n, d = x2.shape
    return pl.pallas_call(
        _matvec_body,
        grid=(n_tc // blk,),
        in_specs=[
            pl.BlockSpec((blk, d), lambda i: (i, 0)),
            pl.BlockSpec((d, 1), lambda i: (0, 0)),
        ],
        out_specs=pl.BlockSpec((blk, 1), lambda i: (i, 0)),
        out_shape=jax.ShapeDtypeStruct((n_tc, 1), jnp.float32),
    )(x2, w2)


def _rne_bf16(v):
    t = v * jnp.float32(65537.0)
    return t - (t - v)


def _sc_logits(x2, w, n_sc, ch=16, unroll=8):
    n, d = x2.shape
    rows = n_sc // _NW
    nch = rows // ch
    nv = d // 16
    mesh = plsc.VectorSubcoreMesh(
        core_axis_name="c", subcore_axis_name="s", num_cores=_NC)

    @functools.partial(
        pl.kernel, mesh=mesh,
        out_type=jax.ShapeDtypeStruct((n_sc,), jnp.float32),
        scratch_types=[
            pltpu.VMEM((d,), jnp.float32),
            pltpu.VMEM((2, ch, d), jnp.float32),
            pltpu.VMEM((rows,), jnp.float32),
            pltpu.SemaphoreType.DMA,
            pltpu.SemaphoreType.DMA,
        ],
    )
    def k(x_hbm, w_hbm, out_hbm, w_v, xbuf, outb, sem0, sem1):
        wid = lax.axis_index("s") * _NC + lax.axis_index("c")
        out_base = wid * rows
        base = (n - n_sc) + out_base
        pltpu.sync_copy(w_hbm, w_v)

        def round_w(i, carry):
            w_v[pl.ds(i * 16, 16)] = _rne_bf16(w_v[pl.ds(i * 16, 16)])
            return carry

        lax.fori_loop(0, nv, round_w, 0)
        lane = lax.iota(jnp.int32, 16)
        sems = (sem0, sem1)

        def start(c):
            return pltpu.async_copy(
                x_hbm.at[pl.ds(base + c * ch, ch), :],
                xbuf.at[c % 2], sems[c % 2])

        pending = start(0)
        for c in range(nch):
            nxt = start(c + 1) if c + 1 < nch else None
            pending.wait()
            pending = nxt

            def do_group(g, carry):
                ov = jnp.zeros((16,), jnp.float32)
                for r16 in range(16):
                    r = g * 16 + r16

                    def dstep(i, acc):
                        b0 = i * (16 * unroll)
                        for u in range(unroll):
                            xv = _rne_bf16(
                                xbuf[c % 2, r, pl.ds(b0 + u * 16, 16)])
                            acc = acc + xv * w_v[pl.ds(b0 + u * 16, 16)]
                        return acc

                    s = lax.fori_loop(
                        0, nv // unroll, dstep,
                        jnp.zeros((16,), jnp.float32))
                    for kk in (8, 4, 2, 1):
                        idx = (lane + jnp.int32(kk)) % jnp.int32(16)
                        s = s + s.at[idx].get(mode="promise_in_bounds")
                    ov = jnp.where(lane == jnp.int32(r16), s, ov)
                outb[pl.ds(c * ch + g * 16, 16)] = ov
                return carry

            lax.fori_loop(0, ch // 16, do_group, 0)

        pltpu.sync_copy(outb, out_hbm.at[pl.ds(out_base, rows)])

    return k(x2, w)


def _make_mask_body(cap):
    def _mask_body(logits_ref, noise_ref, mask_ref, aux_ref):
        lg = logits_ref[...]
        b_, l_ = lg.shape
        noisy = lg + noise_ref[...]
        ui = jax.lax.bitcast_convert_type(noisy, jnp.int32)
        ikey = jnp.where(ui < 0, ui ^ jnp.int32(0x7FFFFFFF), ui)
        msb = jnp.int32(-2147483648)

        def step(i, u):
            cand = u | jax.lax.shift_left(jnp.int32(1), jnp.int32(31) - i)
            cand_s = cand ^ msb
            cnt = jnp.sum((ikey >= cand_s).astype(jnp.int32), axis=1,
                          keepdims=True)
            return jnp.where(cnt >= cap, cand, u)

        u = jax.lax.fori_loop(0, 32, step, jnp.zeros((b_, 1), jnp.int32))
        t = u ^ msb
        gt = ikey > t
        eq = ikey == t
        n_gt = jnp.sum(gt.astype(jnp.int32), axis=1, keepdims=True)
        rem = cap - n_gt
        c = eq.astype(jnp.int32)
        sh = 1
        while sh < l_:
            c = c + jnp.concatenate(
                [jnp.zeros((b_, sh), jnp.int32), c[:, : l_ - sh]], axis=1)
            sh *= 2
        mask = gt | (eq & (c <= rem))
        mask_ref[...] = mask.astype(jnp.int32)

        probs = 1.0 / (1.0 + jnp.exp(-lg))
        rowmean = jnp.sum(probs, axis=1, keepdims=True) * (1.0 / l_)
        dev = rowmean - _CAP_FRAC
        aux_ref[...] = jnp.sum(dev * dev, axis=0, keepdims=True) * (_AUX_W / b_)

    return _mask_body


def _mask_and_aux(logits, noise, cap):
    b, l = logits.shape
    return pl.pallas_call(
        _make_mask_body(cap),
        in_specs=[
            pl.BlockSpec((b, l), lambda: (0, 0)),
            pl.BlockSpec((b, l), lambda: (0, 0)),
        ],
        out_specs=[
            pl.BlockSpec((b, l), lambda: (0, 0)),
            pl.BlockSpec((1, 1), lambda: (0, 0)),
        ],
        out_shape=[
            jax.ShapeDtypeStruct((b, l), jnp.int32),
            jax.ShapeDtypeStruct((1, 1), jnp.float32),
        ],
    )(logits, noise)


_SC_ROWS = 2048


def kernel(x, W):
    b, l, d = x.shape
    cap = max(1, int(l * _CAP_FRAC))
    n = b * l
    x2 = x.reshape(n, d)
    n_tc = n - _SC_ROWS
    logits_tc = _tc_logits(x2, W.reshape(d, 1), n_tc)[:, 0]
    logits_sc = _sc_logits(x2, W.reshape(d), _SC_ROWS)
    logits = jnp.concatenate([logits_tc, logits_sc]).reshape(b, l)
    noise = jax.random.normal(jax.random.key(1), (b, l), jnp.float32) * 0.1
    mask_i, aux = _mask_and_aux(logits, noise, cap)
    return mask_i.astype(jnp.bool_), logits, aux[0, 0]


# speedup vs baseline: 2.6429x; 1.2829x over previous
import jax
import jax.numpy as jnp
from jax.experimental import pallas as pl

_CAP_FRAC = 0.5
_AUX_W = 0.01


def _matvec_body(x_ref, w_ref, out_ref):
    out_ref[...] = jax.lax.dot_general(
        x_ref[...].astype(jnp.bfloat16), w_ref[...].astype(jnp.bfloat16),
        dimension_numbers=(((1,), (0,)), ((), ())),
        preferred_element_type=jnp.float32,
    )


def _compute_logits(x2, w2, blk=1024):
    n, d = x2.shape
    return pl.pallas_call(
        _matvec_body,
        grid=(n // blk,),
        in_specs=[
            pl.BlockSpec((blk, d), lambda i: (i, 0)),
            pl.BlockSpec((d, 1), lambda i: (0, 0)),
        ],
        out_specs=pl.BlockSpec((blk, 1), lambda i: (i, 0)),
        out_shape=jax.ShapeDtypeStruct((n, 1), jnp.float32),
    )(x2, w2)


def _make_mask_body(cap):
    def _mask_body(logits_ref, noise_ref, mask_ref, aux_ref):
        lg = logits_ref[...]
        b_, l_ = lg.shape
        noisy = lg + noise_ref[...]
        ui = jax.lax.bitcast_convert_type(noisy, jnp.int32)
        ikey = jnp.where(ui < 0, ui ^ jnp.int32(0x7FFFFFFF), ui)
        msb = jnp.int32(-2147483648)

        def step(i, u):
            cand = u | jax.lax.shift_left(jnp.int32(1), jnp.int32(31) - i)
            cand_s = cand ^ msb
            cnt = jnp.sum((ikey >= cand_s).astype(jnp.int32), axis=1,
                          keepdims=True)
            return jnp.where(cnt >= cap, cand, u)

        u = jax.lax.fori_loop(0, 32, step, jnp.zeros((b_, 1), jnp.int32))
        t = u ^ msb
        gt = ikey > t
        eq = ikey == t
        n_gt = jnp.sum(gt.astype(jnp.int32), axis=1, keepdims=True)
        rem = cap - n_gt
        c = eq.astype(jnp.int32)
        sh = 1
        while sh < l_:
            c = c + jnp.concatenate(
                [jnp.zeros((b_, sh), jnp.int32), c[:, : l_ - sh]], axis=1)
            sh *= 2
        mask = gt | (eq & (c <= rem))
        mask_ref[...] = mask.astype(jnp.int32)

        probs = 1.0 / (1.0 + jnp.exp(-lg))
        rowmean = jnp.sum(probs, axis=1, keepdims=True) * (1.0 / l_)
        dev = rowmean - _CAP_FRAC
        aux_ref[...] = jnp.sum(dev * dev, axis=0, keepdims=True) * (_AUX_W / b_)

    return _mask_body


def _mask_and_aux(logits, noise, cap):
    b, l = logits.shape
    return pl.pallas_call(
        _make_mask_body(cap),
        in_specs=[
            pl.BlockSpec((b, l), lambda: (0, 0)),
            pl.BlockSpec((b, l), lambda: (0, 0)),
        ],
        out_specs=[
            pl.BlockSpec((b, l), lambda: (0, 0)),
            pl.BlockSpec((1, 1), lambda: (0, 0)),
        ],
        out_shape=[
            jax.ShapeDtypeStruct((b, l), jnp.int32),
            jax.ShapeDtypeStruct((1, 1), jnp.float32),
        ],
    )(logits, noise)


def kernel(x, W):
    b, l, d = x.shape
    cap = max(1, int(l * _CAP_FRAC))
    logits2 = _compute_logits(x.reshape(b * l, d), W.reshape(d, 1))
    logits = logits2.reshape(b, l)
    noise = jax.random.normal(jax.random.key(1), (b, l), jnp.float32) * 0.1
    mask_i, aux = _mask_and_aux(logits, noise, cap)
    return mask_i.astype(jnp.bool_), logits, aux[0, 0]
